# baseline (device time: 42264 ns/iter reference)
import jax
import jax.numpy as jnp
from jax import lax
from jax.experimental import pallas as pl
from jax.experimental.pallas import tpu as pltpu

Z_DEV = 4
B, SQ, SKV, H, D = 8, 1, 512, 8, 64
HK = H * SKV
PACK = 2 * D


def kernel(Q, K, V):
    def body(q_ref, k_hbm, v_hbm, out_ref, kbuf, vbuf, comm,
             kcp_sems, vcp_sems, send_sems, recv_sems):
        my_x = lax.axis_index("x")
        my_y = lax.axis_index("y")
        my_z = lax.axis_index("z")

        barrier_sem = pltpu.get_barrier_semaphore()
        for r in (1, 2, 3):
            pl.semaphore_signal(
                barrier_sem,
                inc=1,
                device_id=(my_x, my_y, (my_z + r) % Z_DEV),
                device_id_type=pl.DeviceIdType.MESH,
            )

        mask = (
            lax.broadcasted_iota(jnp.int32, (H, HK), 1) // SKV
            == lax.broadcasted_iota(jnp.int32, (H, HK), 0)
        ).astype(jnp.float32)

        def copy_in(b, slot):
            cps = []
            for h in range(H):
                cps.append(pltpu.make_async_copy(
                    k_hbm.at[b, :, h, :],
                    kbuf.at[slot, pl.ds(SKV * h, SKV), :],
                    kcp_sems.at[slot],
                ))
                cps.append(pltpu.make_async_copy(
                    v_hbm.at[b, :, h, :],
                    vbuf.at[slot, pl.ds(SKV * h, SKV), :],
                    vcp_sems.at[slot],
                ))
            for cp in cps:
                cp.start()
            return cps

        scale = D ** -0.5
        pend = [copy_in(0, 0), copy_in(1, 1)]
        for b in range(B):
            slot = b % 2
            for cp in pend[0]:
                cp.wait()
            pend = pend[1:]
            q_b = q_ref[b, 0]
            s_all = lax.dot_general(
                q_b, kbuf[slot], (((1,), (1,)), ((), ()))
            )
            s_flat = jnp.sum(s_all * mask, axis=0, keepdims=True)
            p_flat = jnp.exp(s_flat * scale)
            l_col = lax.dot_general(
                mask, p_flat, (((1,), (1,)), ((), ()))
            )
            p8 = p_flat * mask
            o_hd = jax.lax.dot(p8, vbuf[slot])
            comm[0, b] = jnp.concatenate(
                [o_hd, l_col, jnp.zeros((H, PACK - D - 1), jnp.float32)],
                axis=1,
            )
            if b + 2 < B:
                pend.append(copy_in(b + 2, slot))

        pl.semaphore_wait(barrier_sem, Z_DEV - 1)

        sends = []
        for r in (1, 2, 3):
            send = pltpu.make_async_remote_copy(
                src_ref=comm.at[0],
                dst_ref=comm.at[Z_DEV - r],
                send_sem=send_sems.at[r - 1],
                recv_sem=recv_sems.at[Z_DEV - r - 1],
                device_id=(my_x, my_y, (my_z + r) % Z_DEV),
                device_id_type=pl.DeviceIdType.MESH,
            )
            send.start()
            sends.append(send)
        for t in (1, 2, 3):
            recv = pltpu.make_async_remote_copy(
                src_ref=comm.at[0],
                dst_ref=comm.at[t],
                send_sem=send_sems.at[t - 1],
                recv_sem=recv_sems.at[t - 1],
                device_id=(my_x, my_y, my_z),
                device_id_type=pl.DeviceIdType.MESH,
            )
            recv.wait_recv()

        tot = jnp.sum(comm[...], axis=0)
        out3 = tot[:, :, :D] / tot[:, :, D:D + 1]
        out_ref[...] = out3[:, None, :, :]

        for send in sends:
            send.wait_send()

    return pl.pallas_call(
        body,
        out_shape=jax.ShapeDtypeStruct((B, SQ, H, D), jnp.float32),
        in_specs=[
            pl.BlockSpec(memory_space=pltpu.VMEM),
            pl.BlockSpec(memory_space=pl.ANY),
            pl.BlockSpec(memory_space=pl.ANY),
        ],
        out_specs=pl.BlockSpec(memory_space=pltpu.VMEM),
        scratch_shapes=[
            pltpu.VMEM((2, HK, D), jnp.float32),
            pltpu.VMEM((2, HK, D), jnp.float32),
            pltpu.VMEM((Z_DEV, B, H, PACK), jnp.float32),
            pltpu.SemaphoreType.DMA((2,)),
            pltpu.SemaphoreType.DMA((2,)),
            pltpu.SemaphoreType.DMA((Z_DEV - 1,)),
            pltpu.SemaphoreType.DMA((Z_DEV - 1,)),
        ],
        compiler_params=pltpu.CompilerParams(collective_id=0),
    )(Q, K, V)


# device time: 40639 ns/iter; 1.0400x vs baseline; 1.0400x over previous
import jax
import jax.numpy as jnp
from jax import lax
from jax.experimental import pallas as pl
from jax.experimental.pallas import tpu as pltpu

Z_DEV = 4
B, SQ, SKV, H, D = 8, 1, 512, 8, 64
PACK = 2 * D


def kernel(Q, K, V):
    def body(q_ref, k_hbm, v_hbm, out_ref, kbuf, vbuf, comm,
             kcp_sems, vcp_sems, send_sems, recv_sems):
        my_x = lax.axis_index("x")
        my_y = lax.axis_index("y")
        my_z = lax.axis_index("z")

        barrier_sem = pltpu.get_barrier_semaphore()
        for r in (1, 2, 3):
            pl.semaphore_signal(
                barrier_sem,
                inc=1,
                device_id=(my_x, my_y, (my_z + r) % Z_DEV),
                device_id_type=pl.DeviceIdType.MESH,
            )

        eye8 = (
            lax.broadcasted_iota(jnp.int32, (H, H), 0)
            == lax.broadcasted_iota(jnp.int32, (H, H), 1)
        ).astype(jnp.float32)
        ones_col = jnp.full((SKV, 1), 1.0, jnp.float32)

        def copy_in(b, slot):
            kc = pltpu.make_async_copy(k_hbm.at[b], kbuf.at[slot],
                                       kcp_sems.at[slot])
            vc = pltpu.make_async_copy(v_hbm.at[b], vbuf.at[slot],
                                       vcp_sems.at[slot])
            kc.start()
            vc.start()
            return kc, vc

        scale = D ** -0.5
        pend = [copy_in(0, 0), copy_in(1, 1)]
        for b in range(B):
            slot = b % 2
            for cp in pend[0]:
                cp.wait()
            pend = pend[1:]
            q_b = q_ref[b, 0]
            s3 = lax.dot_general(
                q_b, kbuf[slot], (((1,), (2,)), ((), ()))
            )
            s_kh = jnp.sum(s3 * eye8[:, None, :], axis=0)
            p_kh = jnp.exp(s_kh * scale)
            l_col = lax.dot_general(
                p_kh, ones_col, (((0,), (0,)), ((), ()))
            )
            cross = lax.dot_general(
                p_kh, vbuf[slot], (((0,), (0,)), ((), ()))
            )
            o_hd = jnp.sum(cross * eye8[:, :, None], axis=0)
            comm[0, b] = jnp.concatenate(
                [o_hd, l_col, jnp.zeros((H, PACK - D - 1), jnp.float32)],
                axis=1,
            )
            if b + 2 < B:
                pend.append(copy_in(b + 2, slot))

        pl.semaphore_wait(barrier_sem, Z_DEV - 1)

        sends = []
        for r in (1, 2, 3):
            send = pltpu.make_async_remote_copy(
                src_ref=comm.at[0],
                dst_ref=comm.at[Z_DEV - r],
                send_sem=send_sems.at[r - 1],
                recv_sem=recv_sems.at[Z_DEV - r - 1],
                device_id=(my_x, my_y, (my_z + r) % Z_DEV),
                device_id_type=pl.DeviceIdType.MESH,
            )
            send.start()
            sends.append(send)
        for t in (1, 2, 3):
            recv = pltpu.make_async_remote_copy(
                src_ref=comm.at[0],
                dst_ref=comm.at[t],
                send_sem=send_sems.at[t - 1],
                recv_sem=recv_sems.at[t - 1],
                device_id=(my_x, my_y, my_z),
                device_id_type=pl.DeviceIdType.MESH,
            )
            recv.wait_recv()

        tot = jnp.sum(comm[...], axis=0)
        out3 = tot[:, :, :D] / tot[:, :, D:D + 1]
        out_ref[...] = out3[:, None, :, :]

        for send in sends:
            send.wait_send()

    return pl.pallas_call(
        body,
        out_shape=jax.ShapeDtypeStruct((B, SQ, H, D), jnp.float32),
        in_specs=[
            pl.BlockSpec(memory_space=pltpu.VMEM),
            pl.BlockSpec(memory_space=pl.ANY),
            pl.BlockSpec(memory_space=pl.ANY),
        ],
        out_specs=pl.BlockSpec(memory_space=pltpu.VMEM),
        scratch_shapes=[
            pltpu.VMEM((2, SKV, H, D), jnp.float32),
            pltpu.VMEM((2, SKV, H, D), jnp.float32),
            pltpu.VMEM((Z_DEV, B, H, PACK), jnp.float32),
            pltpu.SemaphoreType.DMA((2,)),
            pltpu.SemaphoreType.DMA((2,)),
            pltpu.SemaphoreType.DMA((Z_DEV - 1,)),
            pltpu.SemaphoreType.DMA((Z_DEV - 1,)),
        ],
        compiler_params=pltpu.CompilerParams(collective_id=0),
    )(Q, K, V)


# device time: 35441 ns/iter; 1.1925x vs baseline; 1.1467x over previous
import jax
import jax.numpy as jnp
from jax import lax
from jax.experimental import pallas as pl
from jax.experimental.pallas import tpu as pltpu

Z_DEV = 4
B, SQ, SKV, H, D = 8, 1, 512, 8, 64
PACK = 2 * D


def kernel(Q, K, V):
    def body(q_ref, k_hbm, v_hbm, out_ref, kbuf, vbuf, comm,
             kcp_sems, vcp_sems, send_sems, recv_sems):
        my_x = lax.axis_index("x")
        my_y = lax.axis_index("y")
        my_z = lax.axis_index("z")

        barrier_sem = pltpu.get_barrier_semaphore()
        for r in (1, 2, 3):
            pl.semaphore_signal(
                barrier_sem,
                inc=1,
                device_id=(my_x, my_y, (my_z + r) % Z_DEV),
                device_id_type=pl.DeviceIdType.MESH,
            )

        HK = SKV * H
        mask = (
            lax.broadcasted_iota(jnp.int32, (H, HK), 1) % H
            == lax.broadcasted_iota(jnp.int32, (H, HK), 0)
        ).astype(jnp.float32)

        def copy_in(b, slot):
            kc = pltpu.make_async_copy(k_hbm.at[b], kbuf.at[slot],
                                       kcp_sems.at[slot])
            vc = pltpu.make_async_copy(v_hbm.at[b], vbuf.at[slot],
                                       vcp_sems.at[slot])
            kc.start()
            vc.start()
            return kc, vc

        scale = D ** -0.5
        pend = [copy_in(0, 0), copy_in(1, 1)]
        for b in range(B):
            slot = b % 2
            for cp in pend[0]:
                cp.wait()
            pend = pend[1:]
            q_b = q_ref[b, 0]
            k4 = kbuf[slot].reshape(HK, D)
            v4 = vbuf[slot].reshape(HK, D)
            s_all = lax.dot_general(
                q_b, k4, (((1,), (1,)), ((), ()))
            )
            s_flat = jnp.sum(s_all * mask, axis=0, keepdims=True)
            p_flat = jnp.exp(s_flat * scale)
            l_col = lax.dot_general(
                mask, p_flat, (((1,), (1,)), ((), ()))
            )
            p8 = p_flat * mask
            o_hd = jax.lax.dot(p8, v4)
            comm[0, b] = jnp.concatenate(
                [o_hd, l_col, jnp.zeros((H, PACK - D - 1), jnp.float32)],
                axis=1,
            )
            if b + 2 < B:
                pend.append(copy_in(b + 2, slot))

        pl.semaphore_wait(barrier_sem, Z_DEV - 1)

        sends = []
        for r in (1, 2, 3):
            send = pltpu.make_async_remote_copy(
                src_ref=comm.at[0],
                dst_ref=comm.at[Z_DEV - r],
                send_sem=send_sems.at[r - 1],
                recv_sem=recv_sems.at[Z_DEV - r - 1],
                device_id=(my_x, my_y, (my_z + r) % Z_DEV),
                device_id_type=pl.DeviceIdType.MESH,
            )
            send.start()
            sends.append(send)
        for t in (1, 2, 3):
            recv = pltpu.make_async_remote_copy(
                src_ref=comm.at[0],
                dst_ref=comm.at[t],
                send_sem=send_sems.at[t - 1],
                recv_sem=recv_sems.at[t - 1],
                device_id=(my_x, my_y, my_z),
                device_id_type=pl.DeviceIdType.MESH,
            )
            recv.wait_recv()

        tot = jnp.sum(comm[...], axis=0)
        out3 = tot[:, :, :D] / tot[:, :, D:D + 1]
        out_ref[...] = out3[:, None, :, :]

        for send in sends:
            send.wait_send()

    return pl.pallas_call(
        body,
        out_shape=jax.ShapeDtypeStruct((B, SQ, H, D), jnp.float32),
        in_specs=[
            pl.BlockSpec(memory_space=pltpu.VMEM),
            pl.BlockSpec(memory_space=pl.ANY),
            pl.BlockSpec(memory_space=pl.ANY),
        ],
        out_specs=pl.BlockSpec(memory_space=pltpu.VMEM),
        scratch_shapes=[
            pltpu.VMEM((2, SKV, H, D), jnp.float32),
            pltpu.VMEM((2, SKV, H, D), jnp.float32),
            pltpu.VMEM((Z_DEV, B, H, PACK), jnp.float32),
            pltpu.SemaphoreType.DMA((2,)),
            pltpu.SemaphoreType.DMA((2,)),
            pltpu.SemaphoreType.DMA((Z_DEV - 1,)),
            pltpu.SemaphoreType.DMA((Z_DEV - 1,)),
        ],
        compiler_params=pltpu.CompilerParams(collective_id=0),
    )(Q, K, V)


# device time: 34711 ns/iter; 1.2176x vs baseline; 1.0210x over previous
import jax
import jax.numpy as jnp
from jax import lax
from jax.experimental import pallas as pl
from jax.experimental.pallas import tpu as pltpu

Z_DEV = 4
B, SQ, SKV, H, D = 8, 1, 512, 8, 64
PACK = 2 * D


def kernel(Q, K, V):
    def body(q_ref, k_ref, v_ref, out_ref, comm, send_sems, recv_sems):
        my_x = lax.axis_index("x")
        my_y = lax.axis_index("y")
        my_z = lax.axis_index("z")

        barrier_sem = pltpu.get_barrier_semaphore()
        for r in (1, 2, 3):
            pl.semaphore_signal(
                barrier_sem,
                inc=1,
                device_id=(my_x, my_y, (my_z + r) % Z_DEV),
                device_id_type=pl.DeviceIdType.MESH,
            )

        HK = SKV * H
        mask = (
            lax.broadcasted_iota(jnp.int32, (H, HK), 1) % H
            == lax.broadcasted_iota(jnp.int32, (H, HK), 0)
        ).astype(jnp.float32)

        scale = D ** -0.5
        for b in range(B):
            q_b = q_ref[b, 0]
            k4 = k_ref[b].reshape(HK, D)
            v4 = v_ref[b].reshape(HK, D)
            s_all = lax.dot_general(
                q_b, k4, (((1,), (1,)), ((), ()))
            )
            s_flat = jnp.sum(s_all * mask, axis=0, keepdims=True)
            p_flat = jnp.exp(s_flat * scale)
            l_col = lax.dot_general(
                mask, p_flat, (((1,), (1,)), ((), ()))
            )
            p8 = p_flat * mask
            o_hd = jax.lax.dot(p8, v4)
            comm[0, b] = jnp.concatenate(
                [o_hd, l_col, jnp.zeros((H, PACK - D - 1), jnp.float32)],
                axis=1,
            )

        pl.semaphore_wait(barrier_sem, Z_DEV - 1)

        sends = []
        for r in (1, 2, 3):
            send = pltpu.make_async_remote_copy(
                src_ref=comm.at[0],
                dst_ref=comm.at[Z_DEV - r],
                send_sem=send_sems.at[r - 1],
                recv_sem=recv_sems.at[Z_DEV - r - 1],
                device_id=(my_x, my_y, (my_z + r) % Z_DEV),
                device_id_type=pl.DeviceIdType.MESH,
            )
            send.start()
            sends.append(send)
        for t in (1, 2, 3):
            recv = pltpu.make_async_remote_copy(
                src_ref=comm.at[0],
                dst_ref=comm.at[t],
                send_sem=send_sems.at[t - 1],
                recv_sem=recv_sems.at[t - 1],
                device_id=(my_x, my_y, my_z),
                device_id_type=pl.DeviceIdType.MESH,
            )
            recv.wait_recv()

        tot = jnp.sum(comm[...], axis=0)
        out3 = tot[:, :, :D] / tot[:, :, D:D + 1]
        out_ref[...] = out3[:, None, :, :]

        for send in sends:
            send.wait_send()

    return pl.pallas_call(
        body,
        out_shape=jax.ShapeDtypeStruct((B, SQ, H, D), jnp.float32),
        in_specs=[
            pl.BlockSpec(memory_space=pltpu.VMEM),
            pl.BlockSpec(memory_space=pltpu.VMEM),
            pl.BlockSpec(memory_space=pltpu.VMEM),
        ],
        out_specs=pl.BlockSpec(memory_space=pltpu.VMEM),
        scratch_shapes=[
            pltpu.VMEM((Z_DEV, B, H, PACK), jnp.float32),
            pltpu.SemaphoreType.DMA((Z_DEV - 1,)),
            pltpu.SemaphoreType.DMA((Z_DEV - 1,)),
        ],
        compiler_params=pltpu.CompilerParams(collective_id=0),
    )(Q, K, V)


# device time: 20971 ns/iter; 2.0154x vs baseline; 1.6552x over previous
import jax
import jax.numpy as jnp
from jax import lax
from jax.experimental import pallas as pl
from jax.experimental.pallas import tpu as pltpu

Z_DEV = 4
B, SQ, SKV, H, D = 8, 1, 512, 8, 64
HD = H * D
PACK = HD + 128


def kernel(Q, K, V):
    k2 = K.reshape(B, SKV, HD).astype(jnp.bfloat16)
    v2 = V.reshape(B, SKV, HD).astype(jnp.bfloat16)
    q2 = Q.reshape(B, HD)

    def body(q_ref, k_hbm, v_hbm, out_ref, kbuf, vbuf, comm,
             kcp_sems, vcp_sems, send_sems, recv_sems):
        my_x = lax.axis_index("x")
        my_y = lax.axis_index("y")
        my_z = lax.axis_index("z")

        barrier_sem = pltpu.get_barrier_semaphore()
        for r in (1, 2, 3):
            pl.semaphore_signal(
                barrier_sem,
                inc=1,
                device_id=(my_x, my_y, (my_z + r) % Z_DEV),
                device_id_type=pl.DeviceIdType.MESH,
            )

        ids_hd = lax.broadcasted_iota(jnp.int32, (H, HD), 1) // D
        ids_h = lax.broadcasted_iota(jnp.int32, (H, HD), 0)
        e8 = (ids_hd == ids_h).astype(jnp.float32)
        ones_row = jnp.full((1, SKV), 1.0, jnp.float32)

        def copy_in(b, slot):
            kc = pltpu.make_async_copy(k_hbm.at[b], kbuf.at[slot],
                                       kcp_sems.at[slot])
            vc = pltpu.make_async_copy(v_hbm.at[b], vbuf.at[slot],
                                       vcp_sems.at[slot])
            kc.start()
            vc.start()
            return kc, vc

        scale = D ** -0.5
        pend = [copy_in(0, 0), copy_in(1, 1)]
        rows = []
        lrows = []
        for b in range(B):
            slot = b % 2
            for cp in pend[0]:
                cp.wait()
            pend = pend[1:]
            qm = (e8 * q_ref[b:b + 1, :]).astype(jnp.bfloat16)
            s_t = lax.dot_general(
                qm, kbuf[slot], (((1,), (1,)), ((), ())),
                preferred_element_type=jnp.float32,
            )
            p_t = jnp.exp(s_t * scale)
            lrows.append(lax.dot_general(
                ones_row, p_t, (((1,), (1,)), ((), ())),
                preferred_element_type=jnp.float32,
            ))
            cross = lax.dot_general(
                p_t.astype(jnp.bfloat16), vbuf[slot],
                (((1,), (0,)), ((), ())),
                preferred_element_type=jnp.float32,
            )
            rows.append(jnp.sum(cross * e8, axis=0, keepdims=True))
            if b + 2 < B:
                pend.append(copy_in(b + 2, slot))
        o8 = jnp.concatenate(rows, axis=0)
        l8 = jnp.concatenate(lrows, axis=0)
        comm[0] = jnp.concatenate(
            [o8, l8, jnp.zeros((B, PACK - HD - H), jnp.float32)], axis=1
        )

        pl.semaphore_wait(barrier_sem, Z_DEV - 1)

        sends = []
        for r in (1, 2, 3):
            send = pltpu.make_async_remote_copy(
                src_ref=comm.at[0],
                dst_ref=comm.at[Z_DEV - r],
                send_sem=send_sems.at[r - 1],
                recv_sem=recv_sems.at[Z_DEV - r - 1],
                device_id=(my_x, my_y, (my_z + r) % Z_DEV),
                device_id_type=pl.DeviceIdType.MESH,
            )
            send.start()
            sends.append(send)
        for t in (1, 2, 3):
            recv = pltpu.make_async_remote_copy(
                src_ref=comm.at[0],
                dst_ref=comm.at[t],
                send_sem=send_sems.at[t - 1],
                recv_sem=recv_sems.at[t - 1],
                device_id=(my_x, my_y, my_z),
                device_id_type=pl.DeviceIdType.MESH,
            )
            recv.wait_recv()

        tot = jnp.sum(comm[...], axis=0)
        o_sum = tot[:, :HD]
        l_sum = tot[:, HD:HD + H]
        l_flat = jax.lax.dot(l_sum, e8)
        out_ref[...] = o_sum / l_flat

        for send in sends:
            send.wait_send()

    out = pl.pallas_call(
        body,
        out_shape=jax.ShapeDtypeStruct((B, HD), jnp.float32),
        in_specs=[
            pl.BlockSpec(memory_space=pltpu.VMEM),
            pl.BlockSpec(memory_space=pl.ANY),
            pl.BlockSpec(memory_space=pl.ANY),
        ],
        out_specs=pl.BlockSpec(memory_space=pltpu.VMEM),
        scratch_shapes=[
            pltpu.VMEM((2, SKV, HD), jnp.bfloat16),
            pltpu.VMEM((2, SKV, HD), jnp.bfloat16),
            pltpu.VMEM((Z_DEV, B, PACK), jnp.float32),
            pltpu.SemaphoreType.DMA((2,)),
            pltpu.SemaphoreType.DMA((2,)),
            pltpu.SemaphoreType.DMA((Z_DEV - 1,)),
            pltpu.SemaphoreType.DMA((Z_DEV - 1,)),
        ],
        compiler_params=pltpu.CompilerParams(collective_id=0),
    )(q2, k2, v2)
    return out.reshape(B, SQ, H, D)
